# baseline (speedup 1.0000x reference)
import jax
import jax.numpy as jnp
from jax.experimental import pallas as pl
from jax.experimental.pallas import tpu as pltpu

T = 4096
IN = 768
OUT = 768
E = 64
R = 8
ALPHA = 16.0
SCALING = ALPHA / R

TM = 1024


def _fused_body(x_ref, W_ref, b_ref, Wg_ref, A_ref, B_ref, o_ref,
                Wb_ref, A2d_ref, Bb_ref):
    @pl.when(pl.program_id(0) == 0)
    def _prep():
        Wb_ref[...] = W_ref[...].astype(jnp.bfloat16)
        A2d_ref[...] = (
            jnp.transpose(A_ref[...], (1, 0, 2))
            .reshape(IN, E * R).astype(jnp.bfloat16))
        Bb_ref[...] = B_ref[...].astype(jnp.bfloat16)

    x = x_ref[...]
    logits = jnp.dot(x, Wg_ref[...], preferred_element_type=jnp.float32)
    eidx = jax.lax.broadcasted_iota(jnp.int32, (TM, E), 1)
    m1 = jnp.max(logits, axis=1, keepdims=True)
    a1 = jnp.min(jnp.where(logits == m1, eidx, E), axis=1, keepdims=True)
    masked = jnp.where(eidx == a1, -1e30, logits)
    m2 = jnp.max(masked, axis=1, keepdims=True)
    a2 = jnp.min(jnp.where(masked == m2, eidx, E), axis=1, keepdims=True)
    e2 = jnp.exp(m2 - m1)
    g1 = 1.0 / (1.0 + e2)
    g2 = e2 / (1.0 + e2)
    ef = jax.lax.broadcasted_iota(jnp.int32, (TM, E * R), 1) // R
    w_full = jnp.where(ef == a1, g1, 0.0) + jnp.where(ef == a2, g2, 0.0)
    xb = x.astype(jnp.bfloat16)
    base = jnp.dot(xb, Wb_ref[...], preferred_element_type=jnp.float32)
    H = jnp.dot(xb, A2d_ref[...], preferred_element_type=jnp.float32)
    lo = jnp.dot((H * w_full).astype(jnp.bfloat16), Bb_ref[...],
                 preferred_element_type=jnp.float32)
    o_ref[...] = base + b_ref[...] + SCALING * lo


def kernel(x, W, b, Wg, lora_A, lora_B):
    B2d = lora_B.reshape(E * R, OUT)
    b2 = b.reshape(1, OUT)
    grid = (T // TM,)
    return pl.pallas_call(
        _fused_body,
        grid=grid,
        in_specs=[
            pl.BlockSpec((TM, IN), lambda i: (i, 0)),
            pl.BlockSpec((IN, OUT), lambda i: (0, 0)),
            pl.BlockSpec((1, OUT), lambda i: (0, 0)),
            pl.BlockSpec((IN, E), lambda i: (0, 0)),
            pl.BlockSpec((E, IN, R), lambda i: (0, 0, 0)),
            pl.BlockSpec((E * R, OUT), lambda i: (0, 0)),
        ],
        out_specs=pl.BlockSpec((TM, OUT), lambda i: (i, 0)),
        out_shape=jax.ShapeDtypeStruct((T, OUT), jnp.float32),
        scratch_shapes=[
            pltpu.VMEM((IN, OUT), jnp.bfloat16),
            pltpu.VMEM((IN, E * R), jnp.bfloat16),
            pltpu.VMEM((E * R, OUT), jnp.bfloat16),
        ],
    )(x, W, b2, Wg, lora_A, B2d)


# speedup vs baseline: 1.8222x; 1.8222x over previous
import jax
import jax.numpy as jnp
from jax.experimental import pallas as pl

T = 4096
IN = 768
OUT = 768
E = 64
R = 8
ALPHA = 16.0
SCALING = ALPHA / R

TM = 1024


def _fused_body(x_ref, W_ref, b_ref, Wg_ref, A_ref, B_ref, X_ref, o_ref):
    x = x_ref[...]
    logits = jnp.dot(x, Wg_ref[...], preferred_element_type=jnp.float32)
    m1 = jnp.max(logits, axis=1, keepdims=True)
    hit1 = logits == m1
    masked = jnp.where(hit1, -1e30, logits)
    m2 = jnp.max(masked, axis=1, keepdims=True)
    e2 = jnp.exp(m2 - m1)
    g1 = 1.0 / (1.0 + e2)
    g2 = e2 / (1.0 + e2)
    wE = jnp.where(hit1, g1, 0.0) + jnp.where(masked == m2, g2, 0.0)
    w_full = jnp.dot(wE.astype(jnp.bfloat16), X_ref[...],
                     preferred_element_type=jnp.float32)
    xb = x.astype(jnp.bfloat16)
    base = jnp.dot(xb, W_ref[...], preferred_element_type=jnp.float32)
    H = jnp.dot(xb, A_ref[...], preferred_element_type=jnp.float32)
    lo = jnp.dot((H * w_full).astype(jnp.bfloat16), B_ref[...],
                 preferred_element_type=jnp.float32)
    o_ref[...] = base + b_ref[...] + SCALING * lo


def kernel(x, W, b, Wg, lora_A, lora_B):
    Wb = W.astype(jnp.bfloat16)
    A2d = lora_A.astype(jnp.bfloat16).transpose(1, 0, 2).reshape(IN, E * R)
    B2d = lora_B.astype(jnp.bfloat16).reshape(E * R, OUT)
    b2 = b.reshape(1, OUT)
    expand = (jax.lax.broadcasted_iota(jnp.int32, (E, E * R), 0) ==
              jax.lax.broadcasted_iota(jnp.int32, (E, E * R), 1) // R
              ).astype(jnp.bfloat16)
    grid = (T // TM,)
    return pl.pallas_call(
        _fused_body,
        grid=grid,
        in_specs=[
            pl.BlockSpec((TM, IN), lambda i: (i, 0)),
            pl.BlockSpec((IN, OUT), lambda i: (0, 0)),
            pl.BlockSpec((1, OUT), lambda i: (0, 0)),
            pl.BlockSpec((IN, E), lambda i: (0, 0)),
            pl.BlockSpec((IN, E * R), lambda i: (0, 0)),
            pl.BlockSpec((E * R, OUT), lambda i: (0, 0)),
            pl.BlockSpec((E, E * R), lambda i: (0, 0)),
        ],
        out_specs=pl.BlockSpec((TM, OUT), lambda i: (i, 0)),
        out_shape=jax.ShapeDtypeStruct((T, OUT), jnp.float32),
    )(x, Wb, b2, Wg, A2d, B2d, expand)
